# baseline (speedup 1.0000x reference)
import functools

import jax
import jax.numpy as jnp
from jax.experimental import pallas as pl
from jax.experimental.pallas import tpu as pltpu

_NBUF = 3


def _ring_kernel(a_hbm, e_ref, o_ref, abuf, sems, *, block_m, steps):
    i = pl.program_id(0)

    @pl.when(i == 0)
    def _warmup():
        for b in range(_NBUF):
            pltpu.make_async_copy(
                a_hbm.at[pl.ds(b * block_m, block_m), :],
                abuf.at[b],
                sems.at[b],
            ).start()

    slot = jax.lax.rem(i, _NBUF)
    pltpu.make_async_copy(
        a_hbm.at[pl.ds(i * block_m, block_m), :],
        abuf.at[slot],
        sems.at[slot],
    ).wait()

    acc = jnp.dot(abuf[slot], e_ref[...], preferred_element_type=jnp.float32)
    o_ref[...] = jnp.where(acc >= 0, acc, 0.5 * acc)

    @pl.when(i + _NBUF < steps)
    def _issue_next():
        pltpu.make_async_copy(
            a_hbm.at[pl.ds((i + _NBUF) * block_m, block_m), :],
            abuf.at[slot],
            sems.at[slot],
        ).start()


@functools.partial(jax.jit, static_argnames=("block_m", "interpret"))
def _gcn_ring(adj, embeds, block_m=256, interpret=False):
    m, k = adj.shape
    n = embeds.shape[1]
    steps = m // block_m
    return pl.pallas_call(
        functools.partial(_ring_kernel, block_m=block_m, steps=steps),
        grid=(steps,),
        in_specs=[
            pl.BlockSpec(memory_space=pl.ANY),
            pl.BlockSpec((k, n), lambda i: (0, 0)),
        ],
        out_specs=pl.BlockSpec((block_m, n), lambda i: (i, 0)),
        out_shape=jax.ShapeDtypeStruct((m, n), jnp.float32),
        scratch_shapes=[
            pltpu.VMEM((_NBUF, block_m, k), jnp.float32),
            pltpu.SemaphoreType.DMA((_NBUF,)),
        ],
        compiler_params=pltpu.CompilerParams(
            dimension_semantics=("arbitrary",),
        ),
        interpret=interpret,
    )(adj, embeds)


def kernel(adj, embeds):
    return _gcn_ring(adj, embeds)


# speedup vs baseline: 1.0070x; 1.0070x over previous
import functools

import jax
import jax.numpy as jnp
from jax.experimental import pallas as pl
from jax.experimental.pallas import tpu as pltpu


def _gcn_block_kernel(a1_ref, a2_ref, e1_ref, e2_ref, o_ref):
    acc = jnp.dot(a1_ref[...], e1_ref[...], preferred_element_type=jnp.float32)
    acc += jnp.dot(a2_ref[...], e2_ref[...], preferred_element_type=jnp.float32)
    o_ref[...] = jnp.where(acc >= 0, acc, 0.5 * acc)


@functools.partial(jax.jit, static_argnames=("block_m",))
def _gcn_matmul(adj, embeds, block_m=256):
    m, k = adj.shape
    n = embeds.shape[1]
    kh = k // 2
    return pl.pallas_call(
        _gcn_block_kernel,
        grid=(m // block_m,),
        in_specs=[
            pl.BlockSpec((block_m, kh), lambda i: (i, 0)),
            pl.BlockSpec((block_m, kh), lambda i: (i, 1)),
            pl.BlockSpec((kh, n), lambda i: (0, 0)),
            pl.BlockSpec((kh, n), lambda i: (1, 0)),
        ],
        out_specs=pl.BlockSpec((block_m, n), lambda i: (i, 0)),
        out_shape=jax.ShapeDtypeStruct((m, n), jnp.float32),
        compiler_params=pltpu.CompilerParams(
            dimension_semantics=("parallel",),
        ),
    )(adj, adj, embeds, embeds)


def kernel(adj, embeds):
    return _gcn_matmul(adj, embeds)
